# Initial kernel scaffold; baseline (speedup 1.0000x reference)
import jax
import jax.numpy as jnp
from jax.experimental import pallas as pl


def kernel(x, conv1_w, conv1_b, conv2_w, conv2_b, fc1_w, fc1_b, fc2_w, fc2_b, fc3_w, fc3_b):
    raise NotImplementedError("write your pallas kernel here")


# speedup vs baseline: 89.6710x; 89.6710x over previous
import jax
import jax.numpy as jnp
import numpy as np
from jax.experimental import pallas as pl
from jax.experimental.pallas import tpu as pltpu

_B = 128

_KJ1 = np.clip(np.arange(32)[None, :] - np.arange(28)[:, None], 0, 4)
_M1 = ((np.arange(32)[None, :] - np.arange(28)[:, None] >= 0) &
       (np.arange(32)[None, :] - np.arange(28)[:, None] < 5)).astype(np.float32)
_KJ2 = np.clip(np.arange(14)[None, :] - np.arange(10)[:, None], 0, 4)
_M2 = ((np.arange(14)[None, :] - np.arange(10)[:, None] >= 0) &
       (np.arange(14)[None, :] - np.arange(10)[:, None] < 5)).astype(np.float32)


def _banded_conv1(w):
    wt = w.transpose(0, 2, 3, 1)
    a = wt[:, :, _KJ1, :]
    a = a * _M1[None, None, :, :, None]
    a = a.transpose(2, 0, 1, 3, 4)
    a = jnp.pad(a, ((0, 0), (0, 2), (0, 0), (0, 0), (0, 0)))
    return a.reshape(224, 480).astype(jnp.bfloat16)


def _banded_conv2(w):
    wt = w.transpose(0, 2, 3, 1)
    a = wt[:, :, _KJ2, :]
    a = a * _M2[None, None, :, :, None]
    a = a.transpose(2, 0, 1, 3, 4)
    a = jnp.pad(a, ((0, 0), (0, 0), (0, 0), (0, 0), (0, 2)))
    return a.reshape(160, 560).astype(jnp.bfloat16)


def _lenet_kernel(x_ref, a1_ref, b1_ref, a2_ref, b2_ref,
                  fw1_ref, fb1_ref, fw2_ref, fb2_ref, fw3_ref, fb3_ref,
                  out_ref, p1, p2):
    f32 = jnp.float32
    bf16 = jnp.bfloat16

    a1 = a1_ref[...]
    b1 = b1_ref[...]
    for i in range(14):
        y0 = jnp.dot(a1, x_ref[pl.ds(2 * i * 96, 480), :],
                     preferred_element_type=f32) + b1
        y1 = jnp.dot(a1, x_ref[pl.ds((2 * i + 1) * 96, 480), :],
                     preferred_element_type=f32) + b1
        m = jnp.maximum(jnp.maximum(y0, y1), 0.0)
        mw = jnp.max(m.reshape(14, 2, 8, _B), axis=1)
        p1[pl.ds(i * 112, 112), :] = mw.reshape(112, _B).astype(bf16)

    a2 = a2_ref[...]
    b2 = b2_ref[...]
    for i in range(5):
        y0 = jnp.dot(a2, p1[pl.ds(2 * i * 112, 560), :],
                     preferred_element_type=f32) + b2
        y1 = jnp.dot(a2, p1[pl.ds((2 * i + 1) * 112, 560), :],
                     preferred_element_type=f32) + b2
        m = jnp.maximum(jnp.maximum(y0, y1), 0.0)
        mw = jnp.max(m.reshape(5, 2, 16, _B), axis=1)
        p2[pl.ds(i * 80, 80), :] = mw.reshape(80, _B).astype(bf16)

    h1 = jnp.dot(fw1_ref[...], p2[...],
                 preferred_element_type=f32) + fb1_ref[...]
    h1 = jnp.maximum(h1, 0.0).astype(bf16)
    h2 = jnp.dot(fw2_ref[...], h1,
                 preferred_element_type=f32) + fb2_ref[...]
    h2 = jnp.maximum(h2, 0.0).astype(bf16)
    out_ref[...] = jnp.dot(fw3_ref[...], h2,
                           preferred_element_type=f32) + fb3_ref[...]


def _const_spec(shape):
    zeros = (0,) * len(shape)
    return pl.BlockSpec(shape, lambda n, _z=zeros: _z)


@jax.jit
def _forward(x, conv1_w, conv1_b, conv2_w, conv2_b,
             fc1_w, fc1_b, fc2_w, fc2_b, fc3_w, fc3_b):
    N = x.shape[0]
    npad = (-N) % _B
    xt = x.astype(jnp.bfloat16).transpose(2, 3, 1, 0).reshape(32 * 32 * 3, N)
    if npad:
        xt = jnp.pad(xt, ((0, 0), (0, npad)))
    nb = xt.shape[1] // _B

    a1 = _banded_conv1(conv1_w)
    b1 = jnp.tile(jnp.pad(conv1_b, (0, 2)), 28).reshape(224, 1).astype(jnp.float32)
    a2 = _banded_conv2(conv2_w)
    b2 = jnp.tile(conv2_b, 10).reshape(160, 1).astype(jnp.float32)
    fw1 = fc1_w.reshape(120, 16, 5, 5).transpose(0, 2, 3, 1).reshape(120, 400)
    fw1 = fw1.astype(jnp.bfloat16)
    fb1 = fc1_b.reshape(120, 1).astype(jnp.float32)
    fw2 = fc2_w.astype(jnp.bfloat16)
    fb2 = fc2_b.reshape(84, 1).astype(jnp.float32)
    fw3 = jnp.pad(fc3_w, ((0, 6), (0, 0))).astype(jnp.bfloat16)
    fb3 = jnp.pad(fc3_b, (0, 6)).reshape(16, 1).astype(jnp.float32)
    args = (a1, b1, a2, b2, fw1, fb1, fw2, fb2, fw3, fb3)

    out = pl.pallas_call(
        _lenet_kernel,
        out_shape=jax.ShapeDtypeStruct((16, nb * _B), jnp.float32),
        grid_spec=pltpu.PrefetchScalarGridSpec(
            num_scalar_prefetch=0,
            grid=(nb,),
            in_specs=[pl.BlockSpec((32 * 32 * 3, _B), lambda n: (0, n))] +
                     [_const_spec(a.shape) for a in args],
            out_specs=pl.BlockSpec((16, _B), lambda n: (0, n)),
            scratch_shapes=[
                pltpu.VMEM((14 * 112, _B), jnp.bfloat16),
                pltpu.VMEM((400, _B), jnp.bfloat16),
            ]),
        compiler_params=pltpu.CompilerParams(
            dimension_semantics=("parallel",)),
    )(xt, *args)
    return out[:10, :N].T


def kernel(x, conv1_w, conv1_b, conv2_w, conv2_b,
           fc1_w, fc1_b, fc2_w, fc2_b, fc3_w, fc3_b):
    return _forward(x, conv1_w, conv1_b, conv2_w, conv2_b,
                    fc1_w, fc1_b, fc2_w, fc2_b, fc3_w, fc3_b)


# speedup vs baseline: 97.4121x; 1.0863x over previous
import jax
import jax.numpy as jnp
import numpy as np
from jax.experimental import pallas as pl
from jax.experimental.pallas import tpu as pltpu

_B = 128

def _band_sel(n_out, n_in):
    kj = np.arange(5)[:, None, None]
    ow = np.arange(n_out)[None, :, None]
    iw = np.arange(n_in)[None, None, :]
    return (iw == ow + kj).astype(np.float32)

_S1 = _band_sel(28, 32)
_S2 = _band_sel(10, 14)


def _banded_conv1(w):
    a = jnp.einsum('ackb,bwv->wakvc', w, _S1)
    a = jnp.pad(a, ((0, 0), (0, 2), (0, 0), (0, 0), (0, 0)))
    return a.reshape(224, 480).astype(jnp.bfloat16)


def _banded_conv2(w):
    a = jnp.einsum('ackb,bwv->wakvc', w, _S2)
    a = jnp.pad(a, ((0, 0), (0, 0), (0, 0), (0, 0), (0, 2)))
    return a.reshape(160, 560).astype(jnp.bfloat16)


def _lenet_kernel(x_ref, a1_ref, b1_ref, a2_ref, b2_ref,
                  fw1_ref, fb1_ref, fw2_ref, fb2_ref, fw3_ref, fb3_ref,
                  out_ref, p1, p2):
    f32 = jnp.float32
    bf16 = jnp.bfloat16

    a1 = a1_ref[...]
    b1 = b1_ref[...]
    for i in range(14):
        y0 = jnp.dot(a1, x_ref[pl.ds(2 * i * 96, 480), :],
                     preferred_element_type=f32) + b1
        y1 = jnp.dot(a1, x_ref[pl.ds((2 * i + 1) * 96, 480), :],
                     preferred_element_type=f32) + b1
        m = jnp.maximum(jnp.maximum(y0, y1), 0.0)
        mw = jnp.max(m.reshape(14, 2, 8, _B), axis=1)
        p1[pl.ds(i * 112, 112), :] = mw.reshape(112, _B).astype(bf16)

    a2 = a2_ref[...]
    b2 = b2_ref[...]
    for i in range(5):
        y0 = jnp.dot(a2, p1[pl.ds(2 * i * 112, 560), :],
                     preferred_element_type=f32) + b2
        y1 = jnp.dot(a2, p1[pl.ds((2 * i + 1) * 112, 560), :],
                     preferred_element_type=f32) + b2
        m = jnp.maximum(jnp.maximum(y0, y1), 0.0)
        mw = jnp.max(m.reshape(5, 2, 16, _B), axis=1)
        p2[pl.ds(i * 80, 80), :] = mw.reshape(80, _B).astype(bf16)

    h1 = jnp.dot(fw1_ref[...], p2[...],
                 preferred_element_type=f32) + fb1_ref[...]
    h1 = jnp.maximum(h1, 0.0).astype(bf16)
    h2 = jnp.dot(fw2_ref[...], h1,
                 preferred_element_type=f32) + fb2_ref[...]
    h2 = jnp.maximum(h2, 0.0).astype(bf16)
    logits = jnp.dot(fw3_ref[...], h2,
                     preferred_element_type=f32) + fb3_ref[...]
    out_ref[...] = logits.T


def _const_spec(shape):
    zeros = (0,) * len(shape)
    return pl.BlockSpec(shape, lambda n, _z=zeros: _z)


@jax.jit
def _forward(x, conv1_w, conv1_b, conv2_w, conv2_b,
             fc1_w, fc1_b, fc2_w, fc2_b, fc3_w, fc3_b):
    N = x.shape[0]
    npad = (-N) % _B
    xt = x.astype(jnp.bfloat16).transpose(2, 3, 1, 0).reshape(32 * 32 * 3, N)
    if npad:
        xt = jnp.pad(xt, ((0, 0), (0, npad)))
    nb = xt.shape[1] // _B

    a1 = _banded_conv1(conv1_w)
    b1 = jnp.tile(jnp.pad(conv1_b, (0, 2)), 28).reshape(224, 1).astype(jnp.float32)
    a2 = _banded_conv2(conv2_w)
    b2 = jnp.tile(conv2_b, 10).reshape(160, 1).astype(jnp.float32)
    fw1 = fc1_w.reshape(120, 16, 5, 5).transpose(0, 2, 3, 1).reshape(120, 400)
    fw1 = fw1.astype(jnp.bfloat16)
    fb1 = fc1_b.reshape(120, 1).astype(jnp.float32)
    fw2 = fc2_w.astype(jnp.bfloat16)
    fb2 = fc2_b.reshape(84, 1).astype(jnp.float32)
    fw3 = jnp.pad(fc3_w, ((0, 6), (0, 0))).astype(jnp.bfloat16)
    fb3 = jnp.pad(fc3_b, (0, 6)).reshape(16, 1).astype(jnp.float32)
    args = (a1, b1, a2, b2, fw1, fb1, fw2, fb2, fw3, fb3)

    out = pl.pallas_call(
        _lenet_kernel,
        out_shape=jax.ShapeDtypeStruct((nb * _B, 16), jnp.float32),
        grid_spec=pltpu.PrefetchScalarGridSpec(
            num_scalar_prefetch=0,
            grid=(nb,),
            in_specs=[pl.BlockSpec((32 * 32 * 3, _B), lambda n: (0, n))] +
                     [_const_spec(a.shape) for a in args],
            out_specs=pl.BlockSpec((_B, 16), lambda n: (n, 0)),
            scratch_shapes=[
                pltpu.VMEM((14 * 112, _B), jnp.bfloat16),
                pltpu.VMEM((400, _B), jnp.bfloat16),
            ]),
        compiler_params=pltpu.CompilerParams(
            dimension_semantics=("parallel",)),
    )(xt, *args)
    return out[:N, :10]


def kernel(x, conv1_w, conv1_b, conv2_w, conv2_b,
           fc1_w, fc1_b, fc2_w, fc2_b, fc3_w, fc3_b):
    return _forward(x, conv1_w, conv1_b, conv2_w, conv2_b,
                    fc1_w, fc1_b, fc2_w, fc2_b, fc3_w, fc3_b)


# speedup vs baseline: 130.5375x; 1.3401x over previous
import jax
import jax.numpy as jnp
import numpy as np
from jax.experimental import pallas as pl
from jax.experimental.pallas import tpu as pltpu

_B = 128

def _band_sel(n_out, n_in):
    kj = np.arange(5)[:, None, None]
    ow = np.arange(n_out)[None, :, None]
    iw = np.arange(n_in)[None, None, :]
    return (iw == ow + kj).astype(np.float32)

_S1 = _band_sel(28, 32)
_S2 = _band_sel(10, 14)


def _banded_conv1(w):
    a = jnp.einsum('ackb,bwv->wakcv', w, _S1)
    a = jnp.pad(a, ((0, 0), (0, 2), (0, 0), (0, 0), (0, 0)))
    return a.reshape(224, 480).astype(jnp.bfloat16)


def _banded_conv2(w):
    a = jnp.einsum('ackb,bwv->wakvc', w, _S2)
    a = jnp.pad(a, ((0, 0), (0, 0), (0, 0), (0, 0), (0, 2)))
    return a.reshape(160, 560).astype(jnp.bfloat16)


def _lenet_kernel(x_ref, a1_ref, b1_ref, a2_ref, b2_ref,
                  fw1_ref, fb1_ref, fw2_ref, fb2_ref, fw3_ref, fb3_ref,
                  out_ref, xs, p1, p2):
    f32 = jnp.float32
    bf16 = jnp.bfloat16

    for c in range(3):
        ch = x_ref[:, pl.ds(c * 1024, 1024)].astype(bf16)
        xs[:, c] = ch.T.reshape(32, 32, _B)

    a1 = a1_ref[...]
    b1 = b1_ref[...]
    for i in range(14):
        x0 = xs[pl.ds(2 * i, 5)].reshape(480, _B)
        x1 = xs[pl.ds(2 * i + 1, 5)].reshape(480, _B)
        y0 = jnp.dot(a1, x0, preferred_element_type=f32) + b1
        y1 = jnp.dot(a1, x1, preferred_element_type=f32) + b1
        m = jnp.maximum(jnp.maximum(y0, y1), 0.0)
        mw = jnp.max(m.reshape(14, 2, 8, _B), axis=1)
        p1[pl.ds(i * 112, 112), :] = mw.reshape(112, _B).astype(bf16)

    a2 = a2_ref[...]
    b2 = b2_ref[...]
    for i in range(5):
        y0 = jnp.dot(a2, p1[pl.ds(2 * i * 112, 560), :],
                     preferred_element_type=f32) + b2
        y1 = jnp.dot(a2, p1[pl.ds((2 * i + 1) * 112, 560), :],
                     preferred_element_type=f32) + b2
        m = jnp.maximum(jnp.maximum(y0, y1), 0.0)
        mw = jnp.max(m.reshape(5, 2, 16, _B), axis=1)
        p2[pl.ds(i * 80, 80), :] = mw.reshape(80, _B).astype(bf16)

    h1 = jnp.dot(fw1_ref[...], p2[...],
                 preferred_element_type=f32) + fb1_ref[...]
    h1 = jnp.maximum(h1, 0.0).astype(bf16)
    h2 = jnp.dot(fw2_ref[...], h1,
                 preferred_element_type=f32) + fb2_ref[...]
    h2 = jnp.maximum(h2, 0.0).astype(bf16)
    logits = jnp.dot(fw3_ref[...], h2,
                     preferred_element_type=f32) + fb3_ref[...]
    out_ref[...] = logits.T


def _const_spec(shape):
    zeros = (0,) * len(shape)
    return pl.BlockSpec(shape, lambda n, _z=zeros: _z)


@jax.jit
def _forward(x, conv1_w, conv1_b, conv2_w, conv2_b,
             fc1_w, fc1_b, fc2_w, fc2_b, fc3_w, fc3_b):
    N = x.shape[0]
    npad = (-N) % _B
    xt = x.reshape(N, 3 * 32 * 32)
    if npad:
        xt = jnp.pad(xt, ((0, npad), (0, 0)))
    nb = xt.shape[0] // _B

    a1 = _banded_conv1(conv1_w)
    b1 = jnp.tile(jnp.pad(conv1_b, (0, 2)), 28).reshape(224, 1).astype(jnp.float32)
    a2 = _banded_conv2(conv2_w)
    b2 = jnp.tile(conv2_b, 10).reshape(160, 1).astype(jnp.float32)
    fw1 = fc1_w.reshape(120, 16, 5, 5).transpose(0, 2, 3, 1).reshape(120, 400)
    fw1 = fw1.astype(jnp.bfloat16)
    fb1 = fc1_b.reshape(120, 1).astype(jnp.float32)
    fw2 = fc2_w.astype(jnp.bfloat16)
    fb2 = fc2_b.reshape(84, 1).astype(jnp.float32)
    fw3 = jnp.pad(fc3_w, ((0, 6), (0, 0))).astype(jnp.bfloat16)
    fb3 = jnp.pad(fc3_b, (0, 6)).reshape(16, 1).astype(jnp.float32)
    args = (a1, b1, a2, b2, fw1, fb1, fw2, fb2, fw3, fb3)

    out = pl.pallas_call(
        _lenet_kernel,
        out_shape=jax.ShapeDtypeStruct((nb * _B, 16), jnp.float32),
        grid_spec=pltpu.PrefetchScalarGridSpec(
            num_scalar_prefetch=0,
            grid=(nb,),
            in_specs=[pl.BlockSpec((_B, 32 * 32 * 3), lambda n: (n, 0))] +
                     [_const_spec(a.shape) for a in args],
            out_specs=pl.BlockSpec((_B, 16), lambda n: (n, 0)),
            scratch_shapes=[
                pltpu.VMEM((32, 3, 32, _B), jnp.bfloat16),
                pltpu.VMEM((14 * 112, _B), jnp.bfloat16),
                pltpu.VMEM((400, _B), jnp.bfloat16),
            ]),
        compiler_params=pltpu.CompilerParams(
            dimension_semantics=("parallel",)),
    )(xt, *args)
    return out[:N, :10]


def kernel(x, conv1_w, conv1_b, conv2_w, conv2_b,
           fc1_w, fc1_b, fc2_w, fc2_b, fc3_w, fc3_b):
    return _forward(x, conv1_w, conv1_b, conv2_w, conv2_b,
                    fc1_w, fc1_b, fc2_w, fc2_b, fc3_w, fc3_b)


# speedup vs baseline: 182.1826x; 1.3956x over previous
import jax
import jax.numpy as jnp
import numpy as np
from jax.experimental import pallas as pl
from jax.experimental.pallas import tpu as pltpu

_B = 512

def _band_sel(n_out, n_in):
    kj = np.arange(5)[:, None, None]
    ow = np.arange(n_out)[None, :, None]
    iw = np.arange(n_in)[None, None, :]
    return (iw == ow + kj).astype(np.float32)

_S1 = _band_sel(28, 32)
_S2 = _band_sel(10, 14)


def _banded_conv1(w):
    a = jnp.einsum('ackb,bwv->wakcv', w, _S1)
    a = jnp.pad(a, ((0, 0), (0, 2), (0, 0), (0, 0), (0, 0)))
    return a.reshape(224, 480).astype(jnp.bfloat16)


def _banded_conv2(w):
    a = jnp.einsum('ackb,bwv->wakvc', w, _S2)
    a = jnp.pad(a, ((0, 0), (0, 0), (0, 0), (0, 0), (0, 2)))
    return a.reshape(160, 560).astype(jnp.bfloat16)


def _lenet_kernel(x_ref, a1_ref, b1_ref, a2_ref, b2_ref,
                  fw1_ref, fb1_ref, fw2_ref, fb2_ref, fw3_ref, fb3_ref,
                  out_ref, xs, p1, p2):
    f32 = jnp.float32
    bf16 = jnp.bfloat16

    for c in range(3):
        ch = x_ref[:, pl.ds(c * 1024, 1024)].astype(bf16)
        xs[:, c] = ch.T.reshape(32, 32, _B)

    a1 = a1_ref[...]
    b1 = b1_ref[...]
    for i in range(14):
        x0 = xs[pl.ds(2 * i, 5)].reshape(480, _B)
        x1 = xs[pl.ds(2 * i + 1, 5)].reshape(480, _B)
        y0 = jnp.dot(a1, x0, preferred_element_type=f32) + b1
        y1 = jnp.dot(a1, x1, preferred_element_type=f32) + b1
        m = jnp.maximum(jnp.maximum(y0, y1), 0.0)
        mw = jnp.max(m.reshape(14, 2, 8, _B), axis=1)
        p1[pl.ds(i * 112, 112), :] = mw.reshape(112, _B).astype(bf16)

    a2 = a2_ref[...]
    b2 = b2_ref[...]
    for i in range(5):
        y0 = jnp.dot(a2, p1[pl.ds(2 * i * 112, 560), :],
                     preferred_element_type=f32) + b2
        y1 = jnp.dot(a2, p1[pl.ds((2 * i + 1) * 112, 560), :],
                     preferred_element_type=f32) + b2
        m = jnp.maximum(jnp.maximum(y0, y1), 0.0)
        mw = jnp.max(m.reshape(5, 2, 16, _B), axis=1)
        p2[pl.ds(i * 80, 80), :] = mw.reshape(80, _B).astype(bf16)

    h1 = jnp.dot(fw1_ref[...], p2[...],
                 preferred_element_type=f32) + fb1_ref[...]
    h1 = jnp.maximum(h1, 0.0).astype(bf16)
    h2 = jnp.dot(fw2_ref[...], h1,
                 preferred_element_type=f32) + fb2_ref[...]
    h2 = jnp.maximum(h2, 0.0).astype(bf16)
    logits = jnp.dot(fw3_ref[...], h2,
                     preferred_element_type=f32) + fb3_ref[...]
    out_ref[...] = logits.T


def _const_spec(shape):
    zeros = (0,) * len(shape)
    return pl.BlockSpec(shape, lambda n, _z=zeros: _z)


@jax.jit
def _forward(x, conv1_w, conv1_b, conv2_w, conv2_b,
             fc1_w, fc1_b, fc2_w, fc2_b, fc3_w, fc3_b):
    N = x.shape[0]
    npad = (-N) % _B
    xt = x.reshape(N, 3 * 32 * 32)
    if npad:
        xt = jnp.pad(xt, ((0, npad), (0, 0)))
    nb = xt.shape[0] // _B

    a1 = _banded_conv1(conv1_w)
    b1 = jnp.tile(jnp.pad(conv1_b, (0, 2)), 28).reshape(224, 1).astype(jnp.float32)
    a2 = _banded_conv2(conv2_w)
    b2 = jnp.tile(conv2_b, 10).reshape(160, 1).astype(jnp.float32)
    fw1 = fc1_w.reshape(120, 16, 5, 5).transpose(0, 2, 3, 1).reshape(120, 400)
    fw1 = fw1.astype(jnp.bfloat16)
    fb1 = fc1_b.reshape(120, 1).astype(jnp.float32)
    fw2 = fc2_w.astype(jnp.bfloat16)
    fb2 = fc2_b.reshape(84, 1).astype(jnp.float32)
    fw3 = jnp.pad(fc3_w, ((0, 6), (0, 0))).astype(jnp.bfloat16)
    fb3 = jnp.pad(fc3_b, (0, 6)).reshape(16, 1).astype(jnp.float32)
    args = (a1, b1, a2, b2, fw1, fb1, fw2, fb2, fw3, fb3)

    out = pl.pallas_call(
        _lenet_kernel,
        out_shape=jax.ShapeDtypeStruct((nb * _B, 16), jnp.float32),
        grid_spec=pltpu.PrefetchScalarGridSpec(
            num_scalar_prefetch=0,
            grid=(nb,),
            in_specs=[pl.BlockSpec((_B, 32 * 32 * 3), lambda n: (n, 0))] +
                     [_const_spec(a.shape) for a in args],
            out_specs=pl.BlockSpec((_B, 16), lambda n: (n, 0)),
            scratch_shapes=[
                pltpu.VMEM((32, 3, 32, _B), jnp.bfloat16),
                pltpu.VMEM((14 * 112, _B), jnp.bfloat16),
                pltpu.VMEM((400, _B), jnp.bfloat16),
            ]),
        compiler_params=pltpu.CompilerParams(
            dimension_semantics=("parallel",)),
    )(xt, *args)
    return out[:N, :10]


def kernel(x, conv1_w, conv1_b, conv2_w, conv2_b,
           fc1_w, fc1_b, fc2_w, fc2_b, fc3_w, fc3_b):
    return _forward(x, conv1_w, conv1_b, conv2_w, conv2_b,
                    fc1_w, fc1_b, fc2_w, fc2_b, fc3_w, fc3_b)


# speedup vs baseline: 182.9237x; 1.0041x over previous
import jax
import jax.numpy as jnp
import numpy as np
from jax.experimental import pallas as pl
from jax.experimental.pallas import tpu as pltpu

_B = 1024

def _band_sel(n_out, n_in):
    kj = np.arange(5)[:, None, None]
    ow = np.arange(n_out)[None, :, None]
    iw = np.arange(n_in)[None, None, :]
    return (iw == ow + kj).astype(np.float32)

_S1 = _band_sel(28, 32)
_S2 = _band_sel(10, 14)


def _banded_conv1(w):
    a = jnp.einsum('ackb,bwv->wakcv', w, _S1)
    a = jnp.pad(a, ((0, 0), (0, 2), (0, 0), (0, 0), (0, 0)))
    return a.reshape(224, 480).astype(jnp.bfloat16)


def _banded_conv2(w):
    a = jnp.einsum('ackb,bwv->wakvc', w, _S2)
    a = jnp.pad(a, ((0, 0), (0, 0), (0, 0), (0, 0), (0, 2)))
    return a.reshape(160, 560).astype(jnp.bfloat16)


def _lenet_kernel(x_ref, a1_ref, b1_ref, a2_ref, b2_ref,
                  fw1_ref, fb1_ref, fw2_ref, fb2_ref, fw3_ref, fb3_ref,
                  out_ref, xs, p1, p2):
    f32 = jnp.float32
    bf16 = jnp.bfloat16

    for c in range(3):
        ch = x_ref[:, pl.ds(c * 1024, 1024)].astype(bf16)
        xs[:, c] = ch.T.reshape(32, 32, _B)

    a1 = a1_ref[...]
    b1 = b1_ref[...]
    for i in range(14):
        x0 = xs[pl.ds(2 * i, 5)].reshape(480, _B)
        x1 = xs[pl.ds(2 * i + 1, 5)].reshape(480, _B)
        y0 = jnp.dot(a1, x0, preferred_element_type=f32) + b1
        y1 = jnp.dot(a1, x1, preferred_element_type=f32) + b1
        m = jnp.maximum(jnp.maximum(y0, y1), 0.0)
        mw = jnp.max(m.reshape(14, 2, 8, _B), axis=1)
        p1[pl.ds(i * 112, 112), :] = mw.reshape(112, _B).astype(bf16)

    a2 = a2_ref[...]
    b2 = b2_ref[...]
    for i in range(5):
        y0 = jnp.dot(a2, p1[pl.ds(2 * i * 112, 560), :],
                     preferred_element_type=f32) + b2
        y1 = jnp.dot(a2, p1[pl.ds((2 * i + 1) * 112, 560), :],
                     preferred_element_type=f32) + b2
        m = jnp.maximum(jnp.maximum(y0, y1), 0.0)
        mw = jnp.max(m.reshape(5, 2, 16, _B), axis=1)
        p2[pl.ds(i * 80, 80), :] = mw.reshape(80, _B).astype(bf16)

    h1 = jnp.dot(fw1_ref[...], p2[...],
                 preferred_element_type=f32) + fb1_ref[...]
    h1 = jnp.maximum(h1, 0.0).astype(bf16)
    h2 = jnp.dot(fw2_ref[...], h1,
                 preferred_element_type=f32) + fb2_ref[...]
    h2 = jnp.maximum(h2, 0.0).astype(bf16)
    logits = jnp.dot(fw3_ref[...], h2,
                     preferred_element_type=f32) + fb3_ref[...]
    out_ref[...] = logits.T


def _const_spec(shape):
    zeros = (0,) * len(shape)
    return pl.BlockSpec(shape, lambda n, _z=zeros: _z)


@jax.jit
def _forward(x, conv1_w, conv1_b, conv2_w, conv2_b,
             fc1_w, fc1_b, fc2_w, fc2_b, fc3_w, fc3_b):
    N = x.shape[0]
    npad = (-N) % _B
    xt = x.reshape(N, 3 * 32 * 32)
    if npad:
        xt = jnp.pad(xt, ((0, npad), (0, 0)))
    nb = xt.shape[0] // _B

    a1 = _banded_conv1(conv1_w)
    b1 = jnp.tile(jnp.pad(conv1_b, (0, 2)), 28).reshape(224, 1).astype(jnp.float32)
    a2 = _banded_conv2(conv2_w)
    b2 = jnp.tile(conv2_b, 10).reshape(160, 1).astype(jnp.float32)
    fw1 = fc1_w.reshape(120, 16, 5, 5).transpose(0, 2, 3, 1).reshape(120, 400)
    fw1 = fw1.astype(jnp.bfloat16)
    fb1 = fc1_b.reshape(120, 1).astype(jnp.float32)
    fw2 = fc2_w.astype(jnp.bfloat16)
    fb2 = fc2_b.reshape(84, 1).astype(jnp.float32)
    fw3 = jnp.pad(fc3_w, ((0, 6), (0, 0))).astype(jnp.bfloat16)
    fb3 = jnp.pad(fc3_b, (0, 6)).reshape(16, 1).astype(jnp.float32)
    args = (a1, b1, a2, b2, fw1, fb1, fw2, fb2, fw3, fb3)

    out = pl.pallas_call(
        _lenet_kernel,
        out_shape=jax.ShapeDtypeStruct((nb * _B, 16), jnp.float32),
        grid_spec=pltpu.PrefetchScalarGridSpec(
            num_scalar_prefetch=0,
            grid=(nb,),
            in_specs=[pl.BlockSpec((_B, 32 * 32 * 3), lambda n: (n, 0))] +
                     [_const_spec(a.shape) for a in args],
            out_specs=pl.BlockSpec((_B, 16), lambda n: (n, 0)),
            scratch_shapes=[
                pltpu.VMEM((32, 3, 32, _B), jnp.bfloat16),
                pltpu.VMEM((14 * 112, _B), jnp.bfloat16),
                pltpu.VMEM((400, _B), jnp.bfloat16),
            ]),
        compiler_params=pltpu.CompilerParams(
            dimension_semantics=("parallel",)),
    )(xt, *args)
    return out[:N, :10]


def kernel(x, conv1_w, conv1_b, conv2_w, conv2_b,
           fc1_w, fc1_b, fc2_w, fc2_b, fc3_w, fc3_b):
    return _forward(x, conv1_w, conv1_b, conv2_w, conv2_b,
                    fc1_w, fc1_b, fc2_w, fc2_b, fc3_w, fc3_b)


# speedup vs baseline: 183.8630x; 1.0051x over previous
import jax
import jax.numpy as jnp
import numpy as np
from jax.experimental import pallas as pl
from jax.experimental.pallas import tpu as pltpu

_B = 1024

def _band_sel(n_out, n_in):
    kj = np.arange(5)[:, None, None]
    ow = np.arange(n_out)[None, :, None]
    iw = np.arange(n_in)[None, None, :]
    return (iw == ow + kj).astype(np.float32)

_S1 = _band_sel(28, 32)
_S2 = _band_sel(10, 14)


def _banded_conv1(w):
    a = jnp.einsum('ackb,bwv->wakcv', w, _S1)
    a = jnp.pad(a, ((0, 0), (0, 2), (0, 0), (0, 0), (0, 0)))
    return a.reshape(224, 480).astype(jnp.bfloat16)


def _banded_conv2(w):
    a = jnp.einsum('ackb,bwv->wakvc', w, _S2)
    a = jnp.pad(a, ((0, 0), (0, 0), (0, 0), (0, 0), (0, 2)))
    return a.reshape(160, 560).astype(jnp.bfloat16)


def _lenet_kernel(x_ref, a1_ref, b1_ref, a2_ref, b2_ref,
                  fw1_ref, fb1_ref, fw2_ref, fb2_ref, fw3_ref, fb3_ref,
                  out_ref, xs, p1, p2):
    f32 = jnp.float32
    bf16 = jnp.bfloat16
    b1 = jnp.tile(b1_ref[...], (28, 1))
    b2 = jnp.tile(b2_ref[...], (10, 1))

    for c in range(3):
        ch = x_ref[:, pl.ds(c * 1024, 1024)].astype(bf16)
        xs[:, c] = ch.T.reshape(32, 32, _B)

    a1 = a1_ref[...]
    for i in range(14):
        x0 = xs[pl.ds(2 * i, 5)].reshape(480, _B)
        x1 = xs[pl.ds(2 * i + 1, 5)].reshape(480, _B)
        y0 = jnp.dot(a1, x0, preferred_element_type=f32) + b1
        y1 = jnp.dot(a1, x1, preferred_element_type=f32) + b1
        m = jnp.maximum(jnp.maximum(y0, y1), 0.0)
        mw = jnp.max(m.reshape(14, 2, 8, _B), axis=1)
        p1[pl.ds(i * 112, 112), :] = mw.reshape(112, _B).astype(bf16)

    a2 = a2_ref[...]
    for i in range(5):
        y0 = jnp.dot(a2, p1[pl.ds(2 * i * 112, 560), :],
                     preferred_element_type=f32) + b2
        y1 = jnp.dot(a2, p1[pl.ds((2 * i + 1) * 112, 560), :],
                     preferred_element_type=f32) + b2
        m = jnp.maximum(jnp.maximum(y0, y1), 0.0)
        mw = jnp.max(m.reshape(5, 2, 16, _B), axis=1)
        p2[pl.ds(i * 80, 80), :] = mw.reshape(80, _B).astype(bf16)

    h1 = jnp.dot(fw1_ref[...], p2[...],
                 preferred_element_type=f32) + fb1_ref[...]
    h1 = jnp.maximum(h1, 0.0).astype(bf16)
    h2 = jnp.dot(fw2_ref[...], h1,
                 preferred_element_type=f32) + fb2_ref[...]
    h2 = jnp.maximum(h2, 0.0).astype(bf16)
    logits = jnp.dot(fw3_ref[...], h2,
                     preferred_element_type=f32) + fb3_ref[...]
    out_ref[...] = logits.T


def _const_spec(shape):
    zeros = (0,) * len(shape)
    return pl.BlockSpec(shape, lambda n, _z=zeros: _z)


@jax.jit
def _forward(x, conv1_w, conv1_b, conv2_w, conv2_b,
             fc1_w, fc1_b, fc2_w, fc2_b, fc3_w, fc3_b):
    N = x.shape[0]
    npad = (-N) % _B
    xt = x.reshape(N, 3 * 32 * 32)
    if npad:
        xt = jnp.pad(xt, ((0, npad), (0, 0)))
    nb = xt.shape[0] // _B

    a1 = _banded_conv1(conv1_w)
    b1 = jnp.pad(conv1_b, (0, 2)).reshape(8, 1).astype(jnp.float32)
    a2 = _banded_conv2(conv2_w)
    b2 = conv2_b.reshape(16, 1).astype(jnp.float32)
    fw1 = fc1_w.reshape(120, 16, 5, 5).transpose(0, 2, 3, 1).reshape(120, 400)
    fw1 = fw1.astype(jnp.bfloat16)
    fb1 = fc1_b.reshape(120, 1).astype(jnp.float32)
    fw2 = fc2_w.astype(jnp.bfloat16)
    fb2 = fc2_b.reshape(84, 1).astype(jnp.float32)
    fw3 = fc3_w.astype(jnp.bfloat16)
    fb3 = fc3_b.reshape(10, 1).astype(jnp.float32)
    args = (a1, b1, a2, b2, fw1, fb1, fw2, fb2, fw3, fb3)

    out = pl.pallas_call(
        _lenet_kernel,
        out_shape=jax.ShapeDtypeStruct((nb * _B, 10), jnp.float32),
        grid_spec=pltpu.PrefetchScalarGridSpec(
            num_scalar_prefetch=0,
            grid=(nb,),
            in_specs=[pl.BlockSpec((_B, 32 * 32 * 3), lambda n: (n, 0))] +
                     [_const_spec(a.shape) for a in args],
            out_specs=pl.BlockSpec((_B, 10), lambda n: (n, 0)),
            scratch_shapes=[
                pltpu.VMEM((32, 3, 32, _B), jnp.bfloat16),
                pltpu.VMEM((14 * 112, _B), jnp.bfloat16),
                pltpu.VMEM((400, _B), jnp.bfloat16),
            ]),
        compiler_params=pltpu.CompilerParams(
            dimension_semantics=("parallel",)),
    )(xt, *args)
    return out if npad == 0 else out[:N]


def kernel(x, conv1_w, conv1_b, conv2_w, conv2_b,
           fc1_w, fc1_b, fc2_w, fc2_b, fc3_w, fc3_b):
    return _forward(x, conv1_w, conv1_b, conv2_w, conv2_b,
                    fc1_w, fc1_b, fc2_w, fc2_b, fc3_w, fc3_b)


# speedup vs baseline: 184.8785x; 1.0055x over previous
import jax
import jax.numpy as jnp
import numpy as np
from jax.experimental import pallas as pl
from jax.experimental.pallas import tpu as pltpu

_B = 1024

def _band_sel(n_out, n_in):
    kj = np.arange(5)[:, None, None]
    ow = np.arange(n_out)[None, :, None]
    iw = np.arange(n_in)[None, None, :]
    return (iw == ow + kj).astype(np.float32)

_S1 = _band_sel(28, 32)
_S2 = _band_sel(10, 14)


def _banded_conv1(w):
    a = jnp.einsum('ackb,bwv->wakcv', w, _S1)
    a = jnp.pad(a, ((0, 0), (0, 2), (0, 0), (0, 0), (0, 0)))
    return a.reshape(224, 480).astype(jnp.bfloat16)


def _banded_conv2(w):
    a = jnp.einsum('ackb,bwv->wakvc', w, _S2)
    a = jnp.pad(a, ((0, 0), (0, 0), (0, 0), (0, 0), (0, 2)))
    return a.reshape(160, 560).astype(jnp.bfloat16)


def _lenet_kernel(x_ref, a1_ref, b1_ref, a2_ref, b2_ref,
                  fw1_ref, fb1_ref, fw2_ref, fb2_ref, fw3_ref, fb3_ref,
                  out_ref, xs, p1, p2):
    f32 = jnp.float32
    bf16 = jnp.bfloat16
    b1 = jnp.tile(b1_ref[...], (28, 1))
    b2 = jnp.tile(b2_ref[...], (10, 1))

    for c in range(3):
        ch = x_ref[:, pl.ds(c * 1024, 1024)].astype(bf16)
        xs[:, c] = ch.T.reshape(32, 32, _B)

    a1 = a1_ref[...]
    for i in range(14):
        x0 = xs[pl.ds(2 * i, 5)].reshape(480, _B)
        x1 = xs[pl.ds(2 * i + 1, 5)].reshape(480, _B)
        y0 = jnp.dot(a1, x0, preferred_element_type=f32) + b1
        y1 = jnp.dot(a1, x1, preferred_element_type=f32) + b1
        m = jnp.maximum(jnp.maximum(y0, y1), 0.0)
        mw = jnp.max(m.reshape(14, 2, 8, _B), axis=1)
        p1[pl.ds(i * 112, 112), :] = mw.reshape(112, _B).astype(bf16)

    a2 = a2_ref[...]
    for i in range(5):
        y0 = jnp.dot(a2, p1[pl.ds(2 * i * 112, 560), :],
                     preferred_element_type=f32) + b2
        y1 = jnp.dot(a2, p1[pl.ds((2 * i + 1) * 112, 560), :],
                     preferred_element_type=f32) + b2
        m = jnp.maximum(jnp.maximum(y0, y1), 0.0)
        mw = jnp.max(m.reshape(5, 2, 16, _B), axis=1)
        p2[pl.ds(i * 80, 80), :] = mw.reshape(80, _B).astype(bf16)

    h1 = jnp.dot(fw1_ref[...], p2[...],
                 preferred_element_type=f32) + fb1_ref[...]
    h1 = jnp.maximum(h1, 0.0).astype(bf16)
    h2 = jnp.dot(fw2_ref[...], h1,
                 preferred_element_type=f32) + fb2_ref[...]
    h2 = jnp.maximum(h2, 0.0).astype(bf16)
    logits = jnp.dot(fw3_ref[...], h2,
                     preferred_element_type=f32) + fb3_ref[...]
    out_ref[...] = logits.T


def _const_spec(shape):
    zeros = (0,) * len(shape)
    return pl.BlockSpec(shape, lambda i, j, _z=zeros: _z)


@jax.jit
def _forward(x, conv1_w, conv1_b, conv2_w, conv2_b,
             fc1_w, fc1_b, fc2_w, fc2_b, fc3_w, fc3_b):
    N = x.shape[0]
    npad = (-N) % _B
    xt = x.reshape(N, 3 * 32 * 32)
    if npad:
        xt = jnp.pad(xt, ((0, npad), (0, 0)))
    nb = xt.shape[0] // _B

    a1 = _banded_conv1(conv1_w)
    b1 = jnp.pad(conv1_b, (0, 2)).reshape(8, 1).astype(jnp.float32)
    a2 = _banded_conv2(conv2_w)
    b2 = conv2_b.reshape(16, 1).astype(jnp.float32)
    fw1 = fc1_w.reshape(120, 16, 5, 5).transpose(0, 2, 3, 1).reshape(120, 400)
    fw1 = fw1.astype(jnp.bfloat16)
    fb1 = fc1_b.reshape(120, 1).astype(jnp.float32)
    fw2 = fc2_w.astype(jnp.bfloat16)
    fb2 = fc2_b.reshape(84, 1).astype(jnp.float32)
    fw3 = fc3_w.astype(jnp.bfloat16)
    fb3 = fc3_b.reshape(10, 1).astype(jnp.float32)
    args = (a1, b1, a2, b2, fw1, fb1, fw2, fb2, fw3, fb3)

    out = pl.pallas_call(
        _lenet_kernel,
        out_shape=jax.ShapeDtypeStruct((nb * _B, 10), jnp.float32),
        grid_spec=pltpu.PrefetchScalarGridSpec(
            num_scalar_prefetch=0,
            grid=(2, nb // 2),
            in_specs=[pl.BlockSpec((_B, 32 * 32 * 3),
                                   lambda i, j: (i * (4096 // _B // 2) + j, 0))] +
                     [_const_spec(a.shape) for a in args],
            out_specs=pl.BlockSpec((_B, 10),
                                   lambda i, j: (i * (4096 // _B // 2) + j, 0)),
            scratch_shapes=[
                pltpu.VMEM((32, 3, 32, _B), jnp.bfloat16),
                pltpu.VMEM((14 * 112, _B), jnp.bfloat16),
                pltpu.VMEM((400, _B), jnp.bfloat16),
            ]),
        compiler_params=pltpu.CompilerParams(
            dimension_semantics=("parallel", "parallel")),
    )(xt, *args)
    return out if npad == 0 else out[:N]


def kernel(x, conv1_w, conv1_b, conv2_w, conv2_b,
           fc1_w, fc1_b, fc2_w, fc2_b, fc3_w, fc3_b):
    return _forward(x, conv1_w, conv1_b, conv2_w, conv2_b,
                    fc1_w, fc1_b, fc2_w, fc2_b, fc3_w, fc3_b)


# speedup vs baseline: 186.7071x; 1.0099x over previous
import jax
import jax.numpy as jnp
import numpy as np
from jax.experimental import pallas as pl
from jax.experimental.pallas import tpu as pltpu

_B = 1024

def _band_sel(n_out, n_in):
    kj = np.arange(5)[:, None, None]
    ow = np.arange(n_out)[None, :, None]
    iw = np.arange(n_in)[None, None, :]
    return (iw == ow + kj).astype(np.float32)

_S1 = _band_sel(28, 32)
_S2 = _band_sel(10, 14)


def _banded_conv1(w):
    a = jnp.einsum('ackb,bwv->wakcv', w, _S1)
    a = jnp.pad(a, ((0, 0), (0, 2), (0, 0), (0, 0), (0, 0)))
    return a.reshape(224, 480).astype(jnp.bfloat16)


def _banded_conv2(w):
    a = jnp.einsum('ackb,bwv->wakvc', w, _S2)
    a = jnp.pad(a, ((0, 0), (0, 0), (0, 0), (0, 0), (0, 2)))
    return a.reshape(160, 560).astype(jnp.bfloat16)


def _lenet_kernel(x_ref, a1_ref, b1_ref, a2_ref, b2_ref,
                  fw1_ref, fb1_ref, fw2_ref, fb2_ref, fw3_ref, fb3_ref,
                  out_ref, xs, p1, p2):
    f32 = jnp.float32
    bf16 = jnp.bfloat16
    b1 = jnp.tile(b1_ref[...], (28, 1))
    b2 = jnp.tile(b2_ref[...], (10, 1))

    for c in range(3):
        ch = x_ref[:, pl.ds(c * 1024, 1024)].astype(bf16)
        xs[:, c] = ch.T.reshape(32, 32, _B)

    a1 = a1_ref[...]
    for i in range(14):
        x0 = xs[pl.ds(2 * i, 5)].reshape(480, _B)
        x1 = xs[pl.ds(2 * i + 1, 5)].reshape(480, _B)
        y0 = jnp.dot(a1, x0, preferred_element_type=f32)
        y1 = jnp.dot(a1, x1, preferred_element_type=f32)
        m = jnp.maximum(jnp.maximum(y0, y1) + b1, 0.0)
        mw = jnp.max(m.reshape(14, 2, 8, _B), axis=1)
        p1[pl.ds(i * 112, 112), :] = mw.reshape(112, _B).astype(bf16)

    a2 = a2_ref[...]
    for i in range(5):
        y0 = jnp.dot(a2, p1[pl.ds(2 * i * 112, 560), :],
                     preferred_element_type=f32)
        y1 = jnp.dot(a2, p1[pl.ds((2 * i + 1) * 112, 560), :],
                     preferred_element_type=f32)
        m = jnp.maximum(jnp.maximum(y0, y1) + b2, 0.0)
        mw = jnp.max(m.reshape(5, 2, 16, _B), axis=1)
        p2[pl.ds(i * 80, 80), :] = mw.reshape(80, _B).astype(bf16)

    h1 = jnp.dot(fw1_ref[...], p2[...],
                 preferred_element_type=f32) + fb1_ref[...]
    h1 = jnp.maximum(h1, 0.0).astype(bf16)
    h2 = jnp.dot(fw2_ref[...], h1,
                 preferred_element_type=f32) + fb2_ref[...]
    h2 = jnp.maximum(h2, 0.0).astype(bf16)
    logits = jnp.dot(fw3_ref[...], h2,
                     preferred_element_type=f32) + fb3_ref[...]
    out_ref[...] = logits.T


def _const_spec(shape):
    zeros = (0,) * len(shape)
    return pl.BlockSpec(shape, lambda i, j, _z=zeros: _z)


@jax.jit
def _forward(x, conv1_w, conv1_b, conv2_w, conv2_b,
             fc1_w, fc1_b, fc2_w, fc2_b, fc3_w, fc3_b):
    N = x.shape[0]
    npad = (-N) % _B
    xt = x.reshape(N, 3 * 32 * 32)
    if npad:
        xt = jnp.pad(xt, ((0, npad), (0, 0)))
    nb = xt.shape[0] // _B
    g0 = 2 if nb % 2 == 0 else 1
    g1 = nb // g0

    a1 = _banded_conv1(conv1_w)
    b1 = jnp.pad(conv1_b, (0, 2)).reshape(8, 1).astype(jnp.float32)
    a2 = _banded_conv2(conv2_w)
    b2 = conv2_b.reshape(16, 1).astype(jnp.float32)
    fw1 = fc1_w.reshape(120, 16, 5, 5).transpose(0, 2, 3, 1).reshape(120, 400)
    fw1 = fw1.astype(jnp.bfloat16)
    fb1 = fc1_b.reshape(120, 1).astype(jnp.float32)
    fw2 = fc2_w.astype(jnp.bfloat16)
    fb2 = fc2_b.reshape(84, 1).astype(jnp.float32)
    fw3 = fc3_w.astype(jnp.bfloat16)
    fb3 = fc3_b.reshape(10, 1).astype(jnp.float32)
    args = (a1, b1, a2, b2, fw1, fb1, fw2, fb2, fw3, fb3)

    out = pl.pallas_call(
        _lenet_kernel,
        out_shape=jax.ShapeDtypeStruct((nb * _B, 10), jnp.float32),
        grid_spec=pltpu.PrefetchScalarGridSpec(
            num_scalar_prefetch=0,
            grid=(g0, g1),
            in_specs=[pl.BlockSpec((_B, 32 * 32 * 3),
                                   lambda i, j: (i * g1 + j, 0))] +
                     [_const_spec(a.shape) for a in args],
            out_specs=pl.BlockSpec((_B, 10),
                                   lambda i, j: (i * g1 + j, 0)),
            scratch_shapes=[
                pltpu.VMEM((32, 3, 32, _B), jnp.bfloat16),
                pltpu.VMEM((14 * 112, _B), jnp.bfloat16),
                pltpu.VMEM((400, _B), jnp.bfloat16),
            ]),
        compiler_params=pltpu.CompilerParams(
            dimension_semantics=("parallel", "parallel")),
    )(xt, *args)
    return out if npad == 0 else out[:N]


def kernel(x, conv1_w, conv1_b, conv2_w, conv2_b,
           fc1_w, fc1_b, fc2_w, fc2_b, fc3_w, fc3_b):
    return _forward(x, conv1_w, conv1_b, conv2_w, conv2_b,
                    fc1_w, fc1_b, fc2_w, fc2_b, fc3_w, fc3_b)
